# Initial kernel scaffold; baseline (speedup 1.0000x reference)
import jax
import jax.numpy as jnp
from jax.experimental import pallas as pl


def kernel(A, X, home_mask, emb1_W, emb1_b, emb2_W, emb2_b, rgcn0_W, rgcn0_root, rgcn0_b, lin0a_W, lin0a_b, lin0b_W, lin0b_b, rgcn1_W, rgcn1_root, rgcn1_b, lin1a_W, lin1a_b, lin1b_W, lin1b_b, ln_g, ln_b, phi1_W, phi1_b, phi2_W, phi2_b, rho1_W, rho1_b, rho2_W, rho2_b):
    raise NotImplementedError("write your pallas kernel here")


# speedup vs baseline: 157.7452x; 157.7452x over previous
import jax
import jax.numpy as jnp
from jax.experimental import pallas as pl


def _dot(a, b):
    return jax.lax.dot_general(a, b, (((1,), (0,)), ((), ())),
                               preferred_element_type=jnp.float32)


def _dot_t(a, b):
    return jax.lax.dot_general(a, b, (((0,), (0,)), ((), ())),
                               preferred_element_type=jnp.float32)


def _fused_kernel(A_ref, X_ref, hm_ref,
                  emb1_W, emb1_b, emb2_W, emb2_b,
                  rgcn0_W, rgcn0_root, rgcn0_b, lin0a_W, lin0a_b, lin0b_W, lin0b_b,
                  rgcn1_W, rgcn1_root, rgcn1_b, lin1a_W, lin1a_b, lin1b_W, lin1b_b,
                  ln_g, ln_b, phi1_W, phi1_b, phi2_W, phi2_b, rho1_W, rho1_b,
                  rho2_W, out_ref):
    A = A_ref[0]
    X = X_ref[0]
    hm = hm_ref[0]

    H = jnp.maximum(_dot(X, emb1_W[...]) + emb1_b[...], 0.0)
    H = _dot(H, emb2_W[...]) + emb2_b[...]

    absA = jnp.abs(A)
    m0 = (A < 0.0).astype(jnp.float32)
    m1 = (A > 0.0).astype(jnp.float32)
    cnt0 = jnp.maximum(jnp.sum(m0, axis=0, keepdims=True), 1.0)
    cnt1 = jnp.maximum(jnp.sum(m1, axis=0, keepdims=True), 1.0)

    layers = ((rgcn0_W, rgcn0_root, rgcn0_b, lin0a_W, lin0a_b, lin0b_W, lin0b_b),
              (rgcn1_W, rgcn1_root, rgcn1_b, lin1a_W, lin1a_b, lin1b_W, lin1b_b))

    agg = jnp.zeros_like(H)
    for rW, rroot, rb, laW, lab, lbW, lbb in layers:
        H = H + agg
        mean0 = _dot_t(m0, H) / cnt0.T
        mean1 = _dot_t(m1, H) / cnt1.T
        out = _dot(mean0, rW[0]) + _dot(mean1, rW[1])
        H2 = out + _dot(H, rroot[...]) + rb[...]
        agg = _dot(absA, H2)
        mu = jnp.mean(agg, axis=-1, keepdims=True)
        var = jnp.mean((agg - mu) ** 2, axis=-1, keepdims=True)
        agg = (agg - mu) / jnp.sqrt(var + 1e-5) * ln_g[...] + ln_b[...]
        agg = jnp.maximum(agg, 0.0)
        agg = jnp.maximum(_dot(agg, laW[...]) + lab[...], 0.0)
        agg = _dot(agg, lbW[...]) + lbb[...]

    Hf = H + agg
    p = jnp.maximum(_dot(Hf, phi1_W[...]) + phi1_b[...], 0.0)
    p = jnp.maximum(_dot(p, phi2_W[...]) + phi2_b[...], 0.0)

    hs = _dot(hm, p)
    asum = _dot(1.0 - hm, p)
    h_sc = _dot(jnp.maximum(_dot(hs, rho1_W[...]) + rho1_b[...], 0.0), rho2_W[...])
    a_sc = _dot(jnp.maximum(_dot(asum, rho1_W[...]) + rho1_b[...], 0.0), rho2_W[...])
    out_ref[...] = jnp.broadcast_to(0.5 + 0.5 * jnp.tanh(h_sc - a_sc),
                                    out_ref.shape)


@jax.jit
def kernel(A, X, home_mask, emb1_W, emb1_b, emb2_W, emb2_b,
           rgcn0_W, rgcn0_root, rgcn0_b, lin0a_W, lin0a_b, lin0b_W, lin0b_b,
           rgcn1_W, rgcn1_root, rgcn1_b, lin1a_W, lin1a_b, lin1b_W, lin1b_b,
           ln_g, ln_b, phi1_W, phi1_b, phi2_W, phi2_b, rho1_W, rho1_b,
           rho2_W, rho2_b):
    B, N, Din = X.shape
    D = emb1_W.shape[1]
    PHI = phi1_W.shape[1]
    RHO = rho1_W.shape[1]

    hm = home_mask.astype(jnp.float32).reshape(B, 1, N)
    row = lambda v: v.reshape(1, -1)

    def full(shape):
        return pl.BlockSpec(shape, lambda b: (0,) * len(shape))

    weight_args = (
        (emb1_W, (Din, D)), (row(emb1_b), (1, D)),
        (emb2_W, (D, D)), (row(emb2_b), (1, D)),
        (rgcn0_W, (2, D, D)), (rgcn0_root, (D, D)), (row(rgcn0_b), (1, D)),
        (lin0a_W, (D, D)), (row(lin0a_b), (1, D)),
        (lin0b_W, (D, D)), (row(lin0b_b), (1, D)),
        (rgcn1_W, (2, D, D)), (rgcn1_root, (D, D)), (row(rgcn1_b), (1, D)),
        (lin1a_W, (D, D)), (row(lin1a_b), (1, D)),
        (lin1b_W, (D, D)), (row(lin1b_b), (1, D)),
        (row(ln_g), (1, D)), (row(ln_b), (1, D)),
        (phi1_W, (D, PHI)), (row(phi1_b), (1, PHI)),
        (phi2_W, (PHI, PHI)), (row(phi2_b), (1, PHI)),
        (rho1_W, (PHI, RHO)), (row(rho1_b), (1, RHO)),
        (rho2_W, (RHO, 1)),
    )

    out = pl.pallas_call(
        _fused_kernel,
        grid=(B,),
        in_specs=[
            pl.BlockSpec((1, N, N), lambda b: (b, 0, 0)),
            pl.BlockSpec((1, N, Din), lambda b: (b, 0, 0)),
            pl.BlockSpec((1, 1, N), lambda b: (b, 0, 0)),
        ] + [full(shape) for _, shape in weight_args],
        out_specs=pl.BlockSpec((1, 1, 128), lambda b: (b, 0, 0)),
        out_shape=jax.ShapeDtypeStruct((B, 1, 128), jnp.float32),
    )(A, X, hm, *(arr for arr, _ in weight_args))
    return out[:, 0, 0]
